# baseline (speedup 1.0000x reference)
import functools
import math

import jax
import jax.numpy as jnp
from jax import lax
from jax.experimental import pallas as pl
from jax.experimental.pallas import tpu as pltpu
from jax.experimental.pallas import tpu_sc as plsc

_VOCAB = 1_000_000
_D = 64
_BATCH = 4096
_SEQ = 200
_BFLAT = _BATCH * _SEQ

_NC = 2
_NS = 16
_NW = _NC * _NS
_SEQ_PER_W = _BATCH // _NW
_ROWS_PER_W = _BFLAT // _NW
_SEQ_PER_CHUNK = 2
_CHUNK_ROWS = _SEQ_PER_CHUNK * _SEQ
_N_CHUNKS = _SEQ_PER_W // _SEQ_PER_CHUNK
_IDX_COLS = 100
_IDX_ROWS_PER_CHUNK = _CHUNK_ROWS // _IDX_COLS


def _make_pe(seq_len, emb_dim):
    position = jnp.arange(seq_len, dtype=jnp.float32)[:, None]
    div_term = jnp.exp(
        jnp.arange(0, emb_dim, 2, dtype=jnp.float32)
        * (-math.log(10000.0) / emb_dim))
    pe = jnp.zeros((seq_len, emb_dim), dtype=jnp.float32)
    pe = pe.at[:, 0::2].set(jnp.sin(position * div_term))
    pe = pe.at[:, 1::2].set(jnp.cos(position * div_term))
    return pe


def _emb_body(x_hbm, table_hbm, pe_hbm, out_hbm, idx_v, buf_v, pe_v, sem):
    wid = lax.axis_index("s") * _NC + lax.axis_index("c")
    pltpu.sync_copy(pe_hbm, pe_v)

    row0_w = wid * _ROWS_PER_W
    irow0_w = wid * (_ROWS_PER_W // _IDX_COLS)

    def chunk_body(c, carry):
        row0 = row0_w + c * _CHUNK_ROWS
        irow0 = irow0_w + c * _IDX_ROWS_PER_CHUNK
        pltpu.sync_copy(x_hbm.at[pl.ds(irow0, _IDX_ROWS_PER_CHUNK)], idx_v)
        copies = []
        for j in range(_IDX_ROWS_PER_CHUNK):
            copies.append(pltpu.async_copy(
                table_hbm.at[idx_v.at[j]],
                buf_v.at[pl.ds(j * _IDX_COLS, _IDX_COLS)],
                sem))
        for cp in copies:
            cp.wait()

        def pos_body(p, carry2):
            for d in range(_D // 16):
                pe_vec = pe_v[p, pl.ds(d * 16, 16)]
                for t in range(_SEQ_PER_CHUNK):
                    r = t * _SEQ + p
                    buf_v[r, pl.ds(d * 16, 16)] += pe_vec
            return carry2

        lax.fori_loop(0, _SEQ, pos_body, 0)
        pltpu.sync_copy(buf_v, out_hbm.at[pl.ds(row0, _CHUNK_ROWS)])
        return carry

    lax.fori_loop(0, _N_CHUNKS, chunk_body, 0)


_emb_call = functools.partial(
    pl.kernel,
    out_type=jax.ShapeDtypeStruct((_BFLAT, _D), jnp.float32),
    mesh=plsc.VectorSubcoreMesh(core_axis_name="c", subcore_axis_name="s"),
    scratch_types=[
        pltpu.VMEM((_IDX_ROWS_PER_CHUNK, _IDX_COLS), jnp.int32),
        pltpu.VMEM((_CHUNK_ROWS, _D), jnp.float32),
        pltpu.VMEM((_SEQ, _D), jnp.float32),
        pltpu.SemaphoreType.DMA,
    ],
    compiler_params=pltpu.CompilerParams(use_tc_tiling_on_sc=False),
)(_emb_body)


@jax.jit
def kernel(x, table):
    pe = _make_pe(_SEQ, _D)
    x2 = x.reshape(_BFLAT // _IDX_COLS, _IDX_COLS).astype(jnp.int32)
    out = _emb_call(x2, table, pe)
    return out.reshape(_BATCH, _SEQ, _D)
